# Initial kernel scaffold; baseline (speedup 1.0000x reference)
import functools

import jax
import jax.numpy as jnp
from jax import lax
from jax.experimental import pallas as pl
from jax.experimental.pallas import tpu as pltpu
from jax.experimental.pallas import tpu_sc as plsc

NC = 2
NS = 16
LANES = 16


def _prep_body(rel_ref, k_ref, rhat_ref, h_ref):
    rel = rel_ref[...]
    n = jnp.sqrt(jnp.sum(rel * rel, axis=1, keepdims=True))
    rhat = rel / jnp.maximum(n, 1e-12)
    rhat_ref[...] = rhat
    logit = jnp.dot(rhat, k_ref[...], preferred_element_type=jnp.float32)
    h_ref[...] = jnp.exp(logit - jnp.max(logit, axis=0, keepdims=True))


def _tc_prep(rel_emb, kmat):
    R, D = rel_emb.shape
    return pl.pallas_call(
        _prep_body,
        out_shape=(
            jax.ShapeDtypeStruct((R, D), jnp.float32),
            jax.ShapeDtypeStruct((R, 128), jnp.float32),
        ),
    )(rel_emb, kmat)


def _tail_body(x_ref, gate_ref, proxy_ref, bias_ref, o_ref):
    x = x_ref[0]
    p = proxy_ref[0]
    xn = x / jnp.maximum(jnp.sqrt(jnp.sum(x * x, axis=1, keepdims=True)), 1e-12)
    pn = p / jnp.maximum(jnp.sqrt(jnp.sum(p * p, axis=1, keepdims=True)), 1e-12)
    logits = lax.dot_general(xn, pn, (((1,), (1,)), ((), ())),
                             preferred_element_type=jnp.float32)
    logits = logits - jnp.max(logits, axis=1, keepdims=True)
    ex = jnp.exp(logits)
    att = ex / jnp.sum(ex, axis=1, keepdims=True)
    pf = x - jnp.dot(att, p, preferred_element_type=jnp.float32)
    gr = jax.nn.sigmoid(jnp.dot(pf, gate_ref[0], preferred_element_type=jnp.float32)
                        + bias_ref[0])
    o_ref[0] = gr * x + (1.0 - gr) * pf


def _tc_tail(outs2, gate2, proxy2, bias2):
    _, N, F3 = outs2.shape
    P = proxy2.shape[1]
    BN = 1000
    nb = N // BN
    return pl.pallas_call(
        _tail_body,
        grid=(2, nb),
        in_specs=[
            pl.BlockSpec((1, BN, F3), lambda d, i: (d, i, 0)),
            pl.BlockSpec((1, F3, F3), lambda d, i: (d, 0, 0)),
            pl.BlockSpec((1, P, F3), lambda d, i: (d, 0, 0)),
            pl.BlockSpec((1, 1, F3), lambda d, i: (d, 0, 0)),
        ],
        out_specs=pl.BlockSpec((1, BN, F3), lambda d, i: (d, i, 0)),
        out_shape=jax.ShapeDtypeStruct((2, N, F3), jnp.float32),
    )(outs2, gate2, proxy2, bias2)


def _sc_body(N, E, R, D, C,
             src_h, dst_h, rel_h, tcat_h, rhat_h, h4_h, out_h,
             acc, stats, srcb, dstb, relb, idxb, hbuf,
             xb, rb, ob, statb, statdr, zb, htabA, htabB, gsem, rsem):
    cid = lax.axis_index("c")
    sid = lax.axis_index("s")
    ept = E // NS
    nchunk = ept // C
    npt = N // NS
    RC = 125
    nrc = npt // RC
    tec_lo = sid * ept
    zero16 = jnp.zeros((LANES,), jnp.float32)
    iota16 = lax.iota(jnp.int32, LANES)

    pltpu.sync_copy(h4_h.at[2 * cid], htabA)
    pltpu.sync_copy(h4_h.at[2 * cid + 1], htabB)

    def _zero_rows(ref, rows, width):
        def row(i, _):
            for j in range(width // LANES):
                ref[i, pl.ds(j * LANES, LANES)] = zero16
            return 0
        lax.fori_loop(0, rows, row, 0)

    _zero_rows(zb, 25, D)
    _zero_rows(statdr, 128, LANES)
    for rc in range(nrc):
        rbase = sid * npt + rc * RC
        for q in range(5):
            pltpu.sync_copy(zb.at[pl.ds(0, 25)], acc.at[pl.ds(rbase + q * 25, 25)])
        pltpu.sync_copy(statdr.at[pl.ds(0, RC)], stats.at[pl.ds(rbase, RC)])
    one_col = jnp.where(iota16 == 0, 1.0, 0.0)

    def _statb_row(i, _):
        statb[i, pl.ds(0, LANES)] = one_col
        return 0
    lax.fori_loop(0, C, _statb_row, 0)
    plsc.subcore_barrier()

    def _stats_chunk(ch, _):
        base = tec_lo + ch * C
        pltpu.sync_copy(dst_h.at[pl.ds(base, C)], dstb)
        pltpu.sync_copy(rel_h.at[pl.ds(base, C)], relb)
        for ii in range(C // LANES):
            rel16 = relb[pl.ds(ii * LANES, LANES)]
            ha = plsc.load_gather(htabA, [rel16])
            hb = plsc.load_gather(htabB, [rel16])
            rows = iota16 + (ii * LANES)
            plsc.store_scatter(statb, [rows, jnp.full((LANES,), 1, jnp.int32)], ha)
            plsc.store_scatter(statb, [rows, jnp.full((LANES,), 2, jnp.int32)], hb)
        pltpu.sync_copy(statb, stats.at[dstb], add=True)
        return 0
    lax.fori_loop(0, nchunk, _stats_chunk, 0)
    plsc.subcore_barrier()

    def _load_idx(base):
        pltpu.sync_copy(src_h.at[pl.ds(base, C)], srcb)
        pltpu.sync_copy(dst_h.at[pl.ds(base, C)], dstb)
        pltpu.sync_copy(rel_h.at[pl.ds(base, C)], relb)

    def _pass1_chunk(ch, _):
        base = tec_lo + ch * C
        _load_idx(base)
        ccast = jnp.full((LANES,), 0, jnp.int32) + cid
        for ii in range(C // LANES):
            sl = pl.ds(ii * LANES, LANES)
            sv = srcb[sl]
            rv = relb[sl] + N
            idxb[sl] = sv + ccast * (rv - sv)
        pltpu.async_copy(tcat_h.at[idxb], xb.at[pl.ds(0, C)], gsem).wait()
        pltpu.sync_copy(xb.at[pl.ds(0, C)], acc.at[dstb], add=True)
        return 0

    def _depth_chunk(htab, tbase, ch, _):
        base = tec_lo + ch * C
        _load_idx(base)
        for ii in range(C // LANES):
            sl = pl.ds(ii * LANES, LANES)
            idxb[sl] = srcb[sl] + tbase
            hbuf[sl] = plsc.load_gather(htab, [relb[sl]])
        gx = pltpu.async_copy(out_h.at[idxb], xb.at[pl.ds(0, C)], gsem)
        gr = pltpu.async_copy(rhat_h.at[relb], rb, rsem)
        gx.wait()
        gr.wait()

        def edge(i, _):
            a = hbuf[i]
            dotv = xb[i, pl.ds(0, LANES)] * rb[i, pl.ds(0, LANES)]
            for j in range(1, D // LANES):
                sl = pl.ds(j * LANES, LANES)
                dotv = dotv + xb[i, sl] * rb[i, sl]
            c = jnp.sum(dotv)
            b = -2.0 * a * c
            for j in range(D // LANES):
                sl = pl.ds(j * LANES, LANES)
                ob[i, sl] = a * xb[i, sl] + b * rb[i, sl]
            return 0
        lax.fori_loop(0, C, edge, 0)
        pltpu.sync_copy(ob, acc.at[dstb], add=True)
        return 0

    def _drain(scol, plane_base, zero_after):
        for rc in range(nrc):
            rbase = sid * npt + rc * RC
            pltpu.sync_copy(acc.at[pl.ds(rbase, RC)], xb.at[pl.ds(0, RC)])
            pltpu.sync_copy(stats.at[pl.ds(rbase, RC)], statdr.at[pl.ds(0, RC)])
            if zero_after:
                for q in range(5):
                    pltpu.sync_copy(zb.at[pl.ds(0, 25)],
                                    acc.at[pl.ds(rbase + q * 25, 25)])

            def row(i, _):
                s = statdr[i, scol]
                if scol == 0:
                    g = 1.0 / jnp.maximum(s, 1.0)
                else:
                    g = jnp.where(s > 0.0, 1.0 / s, 0.0)
                for j in range(D // LANES):
                    sl = pl.ds(j * LANES, LANES)
                    z = xb[i, sl] * g
                    xb[i, sl] = 1.0 - 2.0 / (jnp.exp(2.0 * z) + 1.0)
                return 0
            lax.fori_loop(0, RC, row, 0)
            pltpu.sync_copy(xb.at[pl.ds(0, RC)],
                            out_h.at[pl.ds(plane_base + rbase, RC)])
        plsc.subcore_barrier()

    lax.fori_loop(0, nchunk, _pass1_chunk, 0)
    plsc.subcore_barrier()
    plane0 = cid * (3 * N)
    _drain(0, plane0, True)

    lax.fori_loop(0, nchunk, functools.partial(_depth_chunk, htabA, plane0), 0)
    plsc.subcore_barrier()
    _drain(1, plane0 + N, True)

    lax.fori_loop(0, nchunk, functools.partial(_depth_chunk, htabB, plane0 + N), 0)
    plsc.subcore_barrier()
    _drain(2, plane0 + 2 * N, False)


def _sc_passes(src, dst, rel, tcat, rhat, h4, N, E, R, D):
    C = 80
    mesh = plsc.VectorSubcoreMesh(core_axis_name="c", subcore_axis_name="s")
    f32 = jnp.float32
    body = functools.partial(_sc_body, N, E, R, D, C)
    kern = pl.kernel(
        body,
        out_type=jax.ShapeDtypeStruct((6 * N, D), f32),
        mesh=mesh,
        scratch_types=[
            pltpu.VMEM_SHARED((N, D), f32),
            pltpu.VMEM_SHARED((N, LANES), f32),
            pltpu.VMEM((C,), jnp.int32),
            pltpu.VMEM((C,), jnp.int32),
            pltpu.VMEM((C,), jnp.int32),
            pltpu.VMEM((C,), jnp.int32),
            pltpu.VMEM((C,), f32),
            pltpu.VMEM((128, D), f32),
            pltpu.VMEM((C, D), f32),
            pltpu.VMEM((C, D), f32),
            pltpu.VMEM((C, LANES), f32),
            pltpu.VMEM((128, LANES), f32),
            pltpu.VMEM((25, D), f32),
            pltpu.VMEM((R,), f32),
            pltpu.VMEM((R,), f32),
            pltpu.SemaphoreType.DMA,
            pltpu.SemaphoreType.DMA,
        ],
    )
    return kern(src, dst, rel, tcat, rhat, h4)


def kernel(edge_index, edge_rel, ent_emb, rel_emb,
           e_gate, e_proxy, e_bias, e_attn0, e_attn1,
           r_gate, r_proxy, r_bias, r_attn0, r_attn1):
    N, D = ent_emb.shape
    R = rel_emb.shape[0]
    E = edge_rel.shape[0]

    src = edge_index[0].astype(jnp.int32)
    dst = edge_index[1].astype(jnp.int32)
    rel = edge_rel.astype(jnp.int32)

    kmat = jnp.zeros((D, 128), jnp.float32)
    kmat = kmat.at[:, 0:1].set(e_attn0).at[:, 1:2].set(e_attn1)
    kmat = kmat.at[:, 2:3].set(r_attn0).at[:, 3:4].set(r_attn1)

    rhat, hfull = _tc_prep(rel_emb, kmat)
    h4 = jnp.transpose(hfull[:, :4])

    tcat = jnp.concatenate([ent_emb, rel_emb], axis=0)
    planes = _sc_passes(src, dst, rel, tcat, rhat, h4, N, E, R, D)
    planes = planes.reshape(6, N, D)

    outs2 = jnp.stack([
        jnp.concatenate([planes[0], planes[1], planes[2]], axis=1),
        jnp.concatenate([planes[3], planes[4], planes[5]], axis=1),
    ])
    gate2 = jnp.stack([e_gate, r_gate])
    proxy2 = jnp.stack([e_proxy, r_proxy])
    bias2 = jnp.stack([e_bias, r_bias])

    res = _tc_tail(outs2, gate2, proxy2, bias2)
    return jnp.concatenate([res[0], res[1]], axis=-1)


# speedup vs baseline: 5.5248x; 5.5248x over previous
import functools

import jax
import jax.numpy as jnp
from jax import lax
from jax.experimental import pallas as pl
from jax.experimental.pallas import tpu as pltpu
from jax.experimental.pallas import tpu_sc as plsc

NC = 2
NS = 16
LANES = 16


_fori = lax.fori_loop


def _copy(src, dst):
    pltpu.sync_copy(src, dst)


def _gather_rows(table, idx_ref, dst, sem):
    return pltpu.async_copy(table.at[idx_ref], dst, sem)


def _scatter_add_rows(src, table, idx_ref):
    pltpu.sync_copy(src, table.at[idx_ref], add=True)


def _prep_body(rel_ref, k_ref, rhat_ref, h_ref):
    rel = rel_ref[...]
    n = jnp.sqrt(jnp.sum(rel * rel, axis=1, keepdims=True))
    rhat = rel / jnp.maximum(n, 1e-12)
    rhat_ref[...] = rhat
    logit = jnp.dot(rhat, k_ref[...], preferred_element_type=jnp.float32)
    h_ref[...] = jnp.exp(logit - jnp.max(logit, axis=0, keepdims=True))


def _tc_prep(rel_emb, kmat):
    R, D = rel_emb.shape
    return pl.pallas_call(
        _prep_body,
        out_shape=(
            jax.ShapeDtypeStruct((R, D), jnp.float32),
            jax.ShapeDtypeStruct((R, 128), jnp.float32),
        ),
    )(rel_emb, kmat)


def _tail_body(x_ref, gate_ref, proxy_ref, bias_ref, o_ref):
    x = x_ref[0]
    p = proxy_ref[0]
    xn = x / jnp.maximum(jnp.sqrt(jnp.sum(x * x, axis=1, keepdims=True)), 1e-12)
    pn = p / jnp.maximum(jnp.sqrt(jnp.sum(p * p, axis=1, keepdims=True)), 1e-12)
    logits = lax.dot_general(xn, pn, (((1,), (1,)), ((), ())),
                             preferred_element_type=jnp.float32)
    logits = logits - jnp.max(logits, axis=1, keepdims=True)
    ex = jnp.exp(logits)
    att = ex / jnp.sum(ex, axis=1, keepdims=True)
    pf = x - jnp.dot(att, p, preferred_element_type=jnp.float32)
    gr = jax.nn.sigmoid(jnp.dot(pf, gate_ref[0], preferred_element_type=jnp.float32)
                        + bias_ref[0])
    o_ref[0] = gr * x + (1.0 - gr) * pf


def _tc_tail(outs2, gate2, proxy2, bias2):
    _, N, F3 = outs2.shape
    P = proxy2.shape[1]
    BN = 1000
    nb = N // BN
    return pl.pallas_call(
        _tail_body,
        grid=(2, nb),
        in_specs=[
            pl.BlockSpec((1, BN, F3), lambda d, i: (d, i, 0)),
            pl.BlockSpec((1, F3, F3), lambda d, i: (d, 0, 0)),
            pl.BlockSpec((1, P, F3), lambda d, i: (d, 0, 0)),
            pl.BlockSpec((1, 1, F3), lambda d, i: (d, 0, 0)),
        ],
        out_specs=pl.BlockSpec((1, BN, F3), lambda d, i: (d, i, 0)),
        out_shape=jax.ShapeDtypeStruct((2, N, F3), jnp.float32),
    )(outs2, gate2, proxy2, bias2)


def _sc_body(NP, E, R4, D, C,
             src_h, dst_h, rel_h, tcat_h, rhat_h, hflat_h, out_h,
             acc, stats, srcb, dstb, relb, idxb, hbuf,
             xb, rb, statb, statdr, zb, htabA, htabB, gsem, rsem):
    _sc_impl(NP, E, R4, D, C, lax.axis_index("c"), lax.axis_index("s"),
             plsc.subcore_barrier,
             src_h, dst_h, rel_h, tcat_h, rhat_h, hflat_h, out_h,
             acc, stats, srcb, dstb, relb, idxb, hbuf,
             xb, rb, statb, statdr, zb, htabA, htabB, gsem, rsem)


def _sc_impl(NP, E, R4, D, C, cid, sid, barrier,
             src_h, dst_h, rel_h, tcat_h, rhat_h, hflat_h, out_h,
             acc, stats, srcb, dstb, relb, idxb, hbuf,
             xb, rb, statb, statdr, zb, htabA, htabB, gsem, rsem):
    ept = E // NS
    nchunk = ept // C
    npt = NP // NS
    RC = 64
    nrc = npt // RC
    tec_lo = pl.multiple_of(sid * ept, 8)
    zero16 = jnp.zeros((LANES,), jnp.float32)
    iota16 = lax.iota(jnp.int32, LANES)

    _copy(hflat_h.at[pl.ds(pl.multiple_of(cid * 2 * R4, 8), R4)], htabA)
    _copy(hflat_h.at[pl.ds(pl.multiple_of(cid * 2 * R4 + R4, 8), R4)], htabB)

    def _zero_rows(ref, rows, width):
        def row(i, _):
            for j in range(width // LANES):
                ref[i, pl.ds(j * LANES, LANES)] = zero16
            return 0
        _fori(0, rows, row, 0)

    _zero_rows(zb, 32, D)
    _zero_rows(statdr, RC, LANES)
    for rc in range(nrc):
        rbase = pl.multiple_of(sid * npt + rc * RC, 8)
        for q in range(RC // 32):
            _copy(zb.at[pl.ds(0, 32)], acc.at[pl.ds(rbase + q * 32, 32)])
        _copy(statdr.at[pl.ds(0, RC)], stats.at[pl.ds(rbase, RC)])
    one_col = jnp.where(iota16 == 0, 1.0, 0.0)

    def _statb_row(i, _):
        statb[i, pl.ds(0, LANES)] = one_col
        return 0
    _fori(0, C, _statb_row, 0)
    barrier()

    def _stats_chunk(ch, _):
        base = pl.multiple_of(tec_lo + ch * C, 8)
        _copy(dst_h.at[pl.ds(base, C)], dstb)
        _copy(rel_h.at[pl.ds(base, C)], relb)
        for ii in range(C // LANES):
            rel16 = relb[pl.ds(ii * LANES, LANES)]
            ha = plsc.load_gather(htabA, [rel16])
            hb = plsc.load_gather(htabB, [rel16])
            rows = iota16 + (ii * LANES)
            plsc.store_scatter(statb, [rows, jnp.full((LANES,), 1, jnp.int32)], ha)
            plsc.store_scatter(statb, [rows, jnp.full((LANES,), 2, jnp.int32)], hb)
        _scatter_add_rows(statb, stats, dstb)
        return 0
    _fori(0, nchunk, _stats_chunk, 0)
    barrier()

    def _load_idx(base):
        _copy(src_h.at[pl.ds(base, C)], srcb)
        _copy(dst_h.at[pl.ds(base, C)], dstb)
        _copy(rel_h.at[pl.ds(base, C)], relb)

    def _pass1_chunk(ch, _):
        base = pl.multiple_of(tec_lo + ch * C, 8)
        _load_idx(base)
        ccast = jnp.full((LANES,), 0, jnp.int32) + cid
        for ii in range(C // LANES):
            sl = pl.ds(ii * LANES, LANES)
            sv = srcb[sl]
            rv = relb[sl] + NP
            idxb[sl] = sv + ccast * (rv - sv)
        _gather_rows(tcat_h, idxb, xb, gsem).wait()
        _scatter_add_rows(xb, acc, dstb)
        return 0

    def _depth_chunk(htab, tbase, ch, _):
        base = pl.multiple_of(tec_lo + ch * C, 8)
        _load_idx(base)
        for ii in range(C // LANES):
            sl = pl.ds(ii * LANES, LANES)
            idxb[sl] = srcb[sl] + tbase
            hbuf[sl] = plsc.load_gather(htab, [relb[sl]])
        gx = _gather_rows(out_h, idxb, xb, gsem)
        gr = _gather_rows(rhat_h, relb, rb, rsem)
        gx.wait()
        gr.wait()

        def edge(i, _):
            a = hbuf[pl.ds(i, LANES)][0]
            dotv = xb[i, pl.ds(0, LANES)] * rb[i, pl.ds(0, LANES)]
            for j in range(1, D // LANES):
                sl = pl.ds(j * LANES, LANES)
                dotv = dotv + xb[i, sl] * rb[i, sl]
            c = jnp.sum(dotv)
            b = -2.0 * a * c
            for j in range(D // LANES):
                sl = pl.ds(j * LANES, LANES)
                xb[i, sl] = a * xb[i, sl] + b * rb[i, sl]
            return 0
        _fori(0, C, edge, 0)
        _scatter_add_rows(xb, acc, dstb)
        return 0

    def _drain(scol, plane_base, zero_after):
        for rc in range(nrc):
            rbase = pl.multiple_of(sid * npt + rc * RC, 8)
            _copy(acc.at[pl.ds(rbase, RC)], xb.at[pl.ds(0, RC)])
            _copy(stats.at[pl.ds(rbase, RC)], statdr.at[pl.ds(0, RC)])
            if zero_after:
                for q in range(RC // 32):
                    _copy(zb.at[pl.ds(0, 32)],
                          acc.at[pl.ds(rbase + q * 32, 32)])

            def row(i, _):
                srow = statdr[i, pl.ds(0, LANES)]
                if scol == 0:
                    gv = 1.0 / jnp.maximum(srow, 1.0)
                else:
                    gv = jnp.where(srow > 0.0, 1.0 / srow, 0.0)
                g = gv[scol]
                for j in range(D // LANES):
                    sl = pl.ds(j * LANES, LANES)
                    z = xb[i, sl] * g
                    xb[i, sl] = 1.0 - 2.0 / (jnp.exp(2.0 * z) + 1.0)
                return 0
            _fori(0, RC, row, 0)
            _copy(xb.at[pl.ds(0, RC)],
                  out_h.at[pl.ds(pl.multiple_of(plane_base + rbase, 8), RC)])
        barrier()

    _fori(0, nchunk, _pass1_chunk, 0)
    barrier()
    plane0 = pl.multiple_of(cid * (3 * NP), 8)
    _drain(0, plane0, True)

    _fori(0, nchunk, functools.partial(_depth_chunk, htabA, plane0), 0)
    barrier()
    _drain(1, plane0 + NP, True)

    _fori(0, nchunk, functools.partial(_depth_chunk, htabB, plane0 + NP), 0)
    barrier()
    _drain(2, plane0 + 2 * NP, False)


def _sc_passes(src, dst, rel, tcat, rhat, hflat, NP, E, R4, D):
    C = 80
    mesh = plsc.VectorSubcoreMesh(core_axis_name="c", subcore_axis_name="s",
                                  num_cores=NC, num_subcores=NS)
    f32 = jnp.float32
    body = functools.partial(_sc_body, NP, E, R4, D, C)
    kern = pl.kernel(
        body,
        out_type=jax.ShapeDtypeStruct((6 * NP, D), f32),
        mesh=mesh,
        compiler_params=pltpu.CompilerParams(needs_layout_passes=False,
                                             use_tc_tiling_on_sc=False),
        scratch_types=[
            pltpu.VMEM_SHARED((NP, D), f32),
            pltpu.VMEM_SHARED((NP, LANES), f32),
            pltpu.VMEM((C,), jnp.int32),
            pltpu.VMEM((C,), jnp.int32),
            pltpu.VMEM((C,), jnp.int32),
            pltpu.VMEM((C,), jnp.int32),
            pltpu.VMEM((C + LANES,), f32),
            pltpu.VMEM((C, D), f32),
            pltpu.VMEM((C, D), f32),
            pltpu.VMEM((C, LANES), f32),
            pltpu.VMEM((64, LANES), f32),
            pltpu.VMEM((32, D), f32),
            pltpu.VMEM((R4,), f32),
            pltpu.VMEM((R4,), f32),
            pltpu.SemaphoreType.DMA,
            pltpu.SemaphoreType.DMA,
        ],
    )
    return kern(src, dst, rel, tcat, rhat, hflat)


def kernel(edge_index, edge_rel, ent_emb, rel_emb,
           e_gate, e_proxy, e_bias, e_attn0, e_attn1,
           r_gate, r_proxy, r_bias, r_attn0, r_attn1):
    N, D = ent_emb.shape
    R = rel_emb.shape[0]
    E = edge_rel.shape[0]
    NP = ((N + NS * 128 - 1) // (NS * 128)) * (NS * 128)
    R4 = ((R + 1023) // 1024) * 1024

    src = edge_index[0].astype(jnp.int32)
    dst = edge_index[1].astype(jnp.int32)
    rel = edge_rel.astype(jnp.int32)

    kmat = jnp.zeros((D, 128), jnp.float32)
    kmat = kmat.at[:, 0:1].set(e_attn0).at[:, 1:2].set(e_attn1)
    kmat = kmat.at[:, 2:3].set(r_attn0).at[:, 3:4].set(r_attn1)

    rhat, hfull = _tc_prep(rel_emb, kmat)
    hflat = jnp.zeros((4, R4), jnp.float32).at[:, :R].set(
        jnp.transpose(hfull[:, :4])).reshape(-1)

    tcat = jnp.zeros((NP + R, D), jnp.float32)
    tcat = tcat.at[:N].set(ent_emb).at[NP:].set(rel_emb)

    planes = _sc_passes(src, dst, rel, tcat, rhat, hflat, NP, E, R4, D)
    planes = planes.reshape(6, NP, D)[:, :N, :]

    outs2 = jnp.stack([
        jnp.concatenate([planes[0], planes[1], planes[2]], axis=1),
        jnp.concatenate([planes[3], planes[4], planes[5]], axis=1),
    ])
    gate2 = jnp.stack([e_gate, r_gate])
    proxy2 = jnp.stack([e_proxy, r_proxy])
    bias2 = jnp.stack([e_bias, r_bias])

    res = _tc_tail(outs2, gate2, proxy2, bias2)
    return jnp.concatenate([res[0], res[1]], axis=-1)


# speedup vs baseline: 8.1334x; 1.4722x over previous
import functools

import jax
import jax.numpy as jnp
from jax import lax
from jax.experimental import pallas as pl
from jax.experimental.pallas import tpu as pltpu
from jax.experimental.pallas import tpu_sc as plsc

NC = 2
NS = 16
LANES = 16


_fori = lax.fori_loop


def _copy(src, dst):
    pltpu.sync_copy(src, dst)


def _gather_rows(table, idx_ref, dst, sem):
    return pltpu.async_copy(table.at[idx_ref], dst, sem)


def _scatter_add_rows(src, table, idx_ref):
    pltpu.sync_copy(src, table.at[idx_ref], add=True)


def _prep_body(rel_ref, k_ref, rhat_ref, h_ref):
    rel = rel_ref[...]
    n = jnp.sqrt(jnp.sum(rel * rel, axis=1, keepdims=True))
    rhat = rel / jnp.maximum(n, 1e-12)
    rhat_ref[...] = rhat
    logit = jnp.dot(rhat, k_ref[...], preferred_element_type=jnp.float32)
    h_ref[...] = jnp.exp(logit - jnp.max(logit, axis=0, keepdims=True))


def _tc_prep(rel_emb, kmat):
    R, D = rel_emb.shape
    return pl.pallas_call(
        _prep_body,
        out_shape=(
            jax.ShapeDtypeStruct((R, D), jnp.float32),
            jax.ShapeDtypeStruct((R, 128), jnp.float32),
        ),
    )(rel_emb, kmat)


def _tail_body(x_ref, gate_ref, proxy_ref, bias_ref, o_ref):
    x = x_ref[0]
    p = proxy_ref[0]
    xn = x / jnp.maximum(jnp.sqrt(jnp.sum(x * x, axis=1, keepdims=True)), 1e-12)
    pn = p / jnp.maximum(jnp.sqrt(jnp.sum(p * p, axis=1, keepdims=True)), 1e-12)
    logits = lax.dot_general(xn, pn, (((1,), (1,)), ((), ())),
                             preferred_element_type=jnp.float32)
    logits = logits - jnp.max(logits, axis=1, keepdims=True)
    ex = jnp.exp(logits)
    att = ex / jnp.sum(ex, axis=1, keepdims=True)
    pf = x - jnp.dot(att, p, preferred_element_type=jnp.float32)
    gr = jax.nn.sigmoid(jnp.dot(pf, gate_ref[0], preferred_element_type=jnp.float32)
                        + bias_ref[0])
    o_ref[0] = gr * x + (1.0 - gr) * pf


def _tc_tail(outs2, gate2, proxy2, bias2):
    _, N, F3 = outs2.shape
    P = proxy2.shape[1]
    BN = 1000
    nb = N // BN
    return pl.pallas_call(
        _tail_body,
        grid=(2, nb),
        in_specs=[
            pl.BlockSpec((1, BN, F3), lambda d, i: (d, i, 0)),
            pl.BlockSpec((1, F3, F3), lambda d, i: (d, 0, 0)),
            pl.BlockSpec((1, P, F3), lambda d, i: (d, 0, 0)),
            pl.BlockSpec((1, 1, F3), lambda d, i: (d, 0, 0)),
        ],
        out_specs=pl.BlockSpec((1, BN, F3), lambda d, i: (d, i, 0)),
        out_shape=jax.ShapeDtypeStruct((2, N, F3), jnp.float32),
    )(outs2, gate2, proxy2, bias2)


def _sc_body(NP, E, R4, D, C,
             src2_h, dst2_h, rel2_h, tcat_h, rhat_h, hflat_h, out_h,
             acc, stats, srcs, dsts, rels, idxb, idxa, idxc, rela, relc, hbuf,
             xb, rb, statb, statdr, zb, htabA, htabB, gsem, rsem):
    _sc_impl(NP, E, R4, D, C, lax.axis_index("c"), lax.axis_index("s"),
             plsc.subcore_barrier,
             src2_h, dst2_h, rel2_h, tcat_h, rhat_h, hflat_h, out_h,
             acc, stats, srcs, dsts, rels, idxb, idxa, idxc, rela, relc, hbuf,
             xb, rb, statb, statdr, zb, htabA, htabB, gsem, rsem)


def _sc_impl(NP, E, R4, D, C, cid, sid, barrier,
             src2_h, dst2_h, rel2_h, tcat_h, rhat_h, hflat_h, out_h,
             acc, stats, srcs, dsts, rels, idxb, idxa, idxc, rela, relc, hbuf,
             xb, rb, statb, statdr, zb, htabA, htabB, gsem, rsem):
    SUP = 5
    ept = E // NS
    nchunk = ept // C
    nsup = nchunk // SUP
    npt = NP // NS
    RC = 64
    nrc = npt // RC
    NG = C // LANES
    row_lo = sid * nchunk
    zero16 = jnp.zeros((LANES,), jnp.float32)
    iota16 = lax.iota(jnp.int32, LANES)

    _copy(hflat_h.at[pl.ds(pl.multiple_of(cid * 2 * R4, 8), R4)], htabA)
    _copy(hflat_h.at[pl.ds(pl.multiple_of(cid * 2 * R4 + R4, 8), R4)], htabB)

    def _zero_rows(ref, rows, width):
        def row(i, _):
            for j in range(width // LANES):
                ref[i, pl.ds(j * LANES, LANES)] = zero16
            return 0
        _fori(0, rows, row, 0)

    _zero_rows(zb, 32, D)
    _zero_rows(statdr, RC, LANES)
    for rc in range(nrc):
        rbase = pl.multiple_of(sid * npt + rc * RC, 8)
        for q in range(RC // 32):
            _copy(zb.at[pl.ds(0, 32)], acc.at[pl.ds(rbase + q * 32, 32)])
        _copy(statdr.at[pl.ds(0, RC)], stats.at[pl.ds(rbase, RC)])
    one_col = jnp.where(iota16 == 0, 1.0, 0.0)

    def _statb_row(i, _):
        statb[i, pl.ds(0, LANES)] = one_col
        return 0
    _fori(0, C, _statb_row, 0)
    barrier()

    def _load_sup(sc, need_src):
        rb8 = row_lo + sc * SUP
        if need_src:
            _copy(src2_h.at[pl.ds(rb8, SUP)], srcs)
        _copy(dst2_h.at[pl.ds(rb8, SUP)], dsts)
        _copy(rel2_h.at[pl.ds(rb8, SUP)], rels)

    def _stats_sup(sc, _):
        _load_sup(sc, False)

        def chunk(j, _):
            for g in range(NG):
                sl = pl.ds(g * LANES, LANES)
                rel16 = rels[j, sl]
                ha = plsc.load_gather(htabA, [rel16])
                hb = plsc.load_gather(htabB, [rel16])
                rows = iota16 + (g * LANES)
                plsc.store_scatter(statb, [rows, jnp.full((LANES,), 1, jnp.int32)], ha)
                plsc.store_scatter(statb, [rows, jnp.full((LANES,), 2, jnp.int32)], hb)
            _scatter_add_rows(statb, stats, dsts.at[j])
            return 0
        _fori(0, SUP, chunk, 0)
        return 0
    _fori(0, nsup, _stats_sup, 0)
    barrier()

    ccast = jnp.full((LANES,), 0, jnp.int32) + cid

    def _pass1_sup(sc, _):
        _load_sup(sc, True)

        def chunk(j, _):
            for g in range(NG):
                sl = pl.ds(g * LANES, LANES)
                sv = srcs[j, sl]
                rv = rels[j, sl] + NP
                idxb[sl] = sv + ccast * (rv - sv)
            _gather_rows(tcat_h, idxb, xb, gsem).wait()
            _scatter_add_rows(xb, acc, dsts.at[j])
            return 0
        _fori(0, SUP, chunk, 0)
        return 0

    def _depth_sup(htab, tbase, sc, _):
        _load_sup(sc, True)

        HG = 3
        def chunk(j, _):
            for g in range(NG):
                sl = pl.ds(g * LANES, LANES)
                gsl = pl.ds((g * LANES) % (HG * LANES), LANES)
                ib = idxa if g < HG else idxc
                rf = rela if g < HG else relc
                ib[gsl] = srcs[j, sl] + tbase
                rf[gsl] = rels[j, sl]
                hbuf[sl] = plsc.load_gather(htab, [rels[j, sl]])
            gxA = _gather_rows(out_h, idxa, xb.at[pl.ds(0, HG * LANES)], gsem)
            grA = _gather_rows(rhat_h, rela, rb.at[pl.ds(0, HG * LANES)], rsem)
            gxB = _gather_rows(out_h, idxc,
                               xb.at[pl.ds(HG * LANES, C - HG * LANES)], gsem)
            grB = _gather_rows(rhat_h, relc,
                               rb.at[pl.ds(HG * LANES, C - HG * LANES)], rsem)
            gxA.wait()
            grA.wait()

            def group(gi, _):
                @pl.when(gi == HG)
                def _wait_second_half():
                    gxB.wait()
                    grB.wait()
                a16 = hbuf[pl.ds(gi * LANES, LANES)]
                for l in range(LANES):
                    i = gi * LANES + l
                    a = a16[l]
                    dotv = xb[i, pl.ds(0, LANES)] * rb[i, pl.ds(0, LANES)]
                    for jj in range(1, D // LANES):
                        sl = pl.ds(jj * LANES, LANES)
                        dotv = dotv + xb[i, sl] * rb[i, sl]
                    b = -2.0 * a * jnp.sum(dotv)
                    for jj in range(D // LANES):
                        sl = pl.ds(jj * LANES, LANES)
                        xb[i, sl] = a * xb[i, sl] + b * rb[i, sl]
                return 0
            _fori(0, NG, group, 0)
            _scatter_add_rows(xb, acc, dsts.at[j])
            return 0
        _fori(0, SUP, chunk, 0)
        return 0

    def _drain(scol, plane_base, zero_after):
        for rc in range(nrc):
            rbase = pl.multiple_of(sid * npt + rc * RC, 8)
            _copy(acc.at[pl.ds(rbase, RC)], xb.at[pl.ds(0, RC)])
            _copy(stats.at[pl.ds(rbase, RC)], statdr.at[pl.ds(0, RC)])
            if zero_after:
                for q in range(RC // 32):
                    _copy(zb.at[pl.ds(0, 32)],
                          acc.at[pl.ds(rbase + q * 32, 32)])

            def row(i, _):
                srow = statdr[i, pl.ds(0, LANES)]
                if scol == 0:
                    gv = 1.0 / jnp.maximum(srow, 1.0)
                else:
                    gv = jnp.where(srow > 0.0, 1.0 / srow, 0.0)
                g = gv[scol]
                for j in range(D // LANES):
                    sl = pl.ds(j * LANES, LANES)
                    z = xb[i, sl] * g
                    xb[i, sl] = 1.0 - 2.0 / (jnp.exp(2.0 * z) + 1.0)
                return 0
            _fori(0, RC, row, 0)
            _copy(xb.at[pl.ds(0, RC)],
                  out_h.at[pl.ds(pl.multiple_of(plane_base + rbase, 8), RC)])
        barrier()

    _fori(0, nsup, _pass1_sup, 0)
    barrier()
    plane0 = pl.multiple_of(cid * (3 * NP), 8)
    _drain(0, plane0, True)

    _fori(0, nsup, functools.partial(_depth_sup, htabA, plane0), 0)
    barrier()
    _drain(1, plane0 + NP, True)

    _fori(0, nsup, functools.partial(_depth_sup, htabB, plane0 + NP), 0)
    barrier()
    _drain(2, plane0 + 2 * NP, False)


def _sc_passes(src2, dst2, rel2, tcat, rhat, hflat, NP, E, R4, D):
    C = src2.shape[1]
    SUP = 5
    mesh = plsc.VectorSubcoreMesh(core_axis_name="c", subcore_axis_name="s",
                                  num_cores=NC, num_subcores=NS)
    f32 = jnp.float32
    i32 = jnp.int32
    body = functools.partial(_sc_body, NP, E, R4, D, C)
    kern = pl.kernel(
        body,
        out_type=jax.ShapeDtypeStruct((6 * NP, D), f32),
        mesh=mesh,
        compiler_params=pltpu.CompilerParams(needs_layout_passes=False,
                                             use_tc_tiling_on_sc=False),
        scratch_types=[
            pltpu.VMEM_SHARED((NP, D), f32),
            pltpu.VMEM_SHARED((NP, LANES), f32),
            pltpu.VMEM((SUP, C), i32),
            pltpu.VMEM((SUP, C), i32),
            pltpu.VMEM((SUP, C), i32),
            pltpu.VMEM((C,), i32),
            pltpu.VMEM((48,), i32),
            pltpu.VMEM((32,), i32),
            pltpu.VMEM((48,), i32),
            pltpu.VMEM((32,), i32),
            pltpu.VMEM((C + LANES,), f32),
            pltpu.VMEM((C, D), f32),
            pltpu.VMEM((C, D), f32),
            pltpu.VMEM((C, LANES), f32),
            pltpu.VMEM((64, LANES), f32),
            pltpu.VMEM((32, D), f32),
            pltpu.VMEM((R4,), f32),
            pltpu.VMEM((R4,), f32),
            pltpu.SemaphoreType.DMA,
            pltpu.SemaphoreType.DMA,
        ],
    )
    return kern(src2, dst2, rel2, tcat, rhat, hflat)


def kernel(edge_index, edge_rel, ent_emb, rel_emb,
           e_gate, e_proxy, e_bias, e_attn0, e_attn1,
           r_gate, r_proxy, r_bias, r_attn0, r_attn1):
    N, D = ent_emb.shape
    R = rel_emb.shape[0]
    E = edge_rel.shape[0]
    NP = ((N + NS * 128 - 1) // (NS * 128)) * (NS * 128)
    R4 = ((R + 1023) // 1024) * 1024

    C = 80
    src2 = edge_index[0].astype(jnp.int32).reshape(E // C, C)
    dst2 = edge_index[1].astype(jnp.int32).reshape(E // C, C)
    rel2 = edge_rel.astype(jnp.int32).reshape(E // C, C)

    kmat = jnp.zeros((D, 128), jnp.float32)
    kmat = kmat.at[:, 0:1].set(e_attn0).at[:, 1:2].set(e_attn1)
    kmat = kmat.at[:, 2:3].set(r_attn0).at[:, 3:4].set(r_attn1)

    rhat, hfull = _tc_prep(rel_emb, kmat)
    hflat = jnp.zeros((4, R4), jnp.float32).at[:, :R].set(
        jnp.transpose(hfull[:, :4])).reshape(-1)

    tcat = jnp.zeros((NP + R, D), jnp.float32)
    tcat = tcat.at[:N].set(ent_emb).at[NP:].set(rel_emb)

    planes = _sc_passes(src2, dst2, rel2, tcat, rhat, hflat, NP, E, R4, D)
    planes = planes.reshape(6, NP, D)[:, :N, :]

    outs2 = jnp.stack([
        jnp.concatenate([planes[0], planes[1], planes[2]], axis=1),
        jnp.concatenate([planes[3], planes[4], planes[5]], axis=1),
    ])
    gate2 = jnp.stack([e_gate, r_gate])
    proxy2 = jnp.stack([e_proxy, r_proxy])
    bias2 = jnp.stack([e_bias, r_bias])

    res = _tc_tail(outs2, gate2, proxy2, bias2)
    return jnp.concatenate([res[0], res[1]], axis=-1)


# speedup vs baseline: 8.3688x; 1.0289x over previous
import functools

import jax
import jax.numpy as jnp
from jax import lax
from jax.experimental import pallas as pl
from jax.experimental.pallas import tpu as pltpu
from jax.experimental.pallas import tpu_sc as plsc

NC = 2
NS = 16
LANES = 16


_fori = lax.fori_loop


def _copy(src, dst):
    pltpu.sync_copy(src, dst)


def _gather_rows(table, idx_ref, dst, sem):
    return pltpu.async_copy(table.at[idx_ref], dst, sem)


def _scatter_add_rows(src, table, idx_ref):
    pltpu.sync_copy(src, table.at[idx_ref], add=True)


def _prep_body(rel_ref, k_ref, rhat_ref, h_ref):
    rel = rel_ref[...]
    n = jnp.sqrt(jnp.sum(rel * rel, axis=1, keepdims=True))
    rhat = rel / jnp.maximum(n, 1e-12)
    rhat_ref[...] = rhat
    logit = jnp.dot(rhat, k_ref[...], preferred_element_type=jnp.float32)
    h_ref[...] = jnp.exp(logit - jnp.max(logit, axis=0, keepdims=True))


def _tc_prep(rel_emb, kmat):
    R, D = rel_emb.shape
    return pl.pallas_call(
        _prep_body,
        out_shape=(
            jax.ShapeDtypeStruct((R, D), jnp.float32),
            jax.ShapeDtypeStruct((R, 128), jnp.float32),
        ),
    )(rel_emb, kmat)


def _tail_body(x_ref, gate_ref, proxy_ref, bias_ref, o_ref):
    x = x_ref[0]
    p = proxy_ref[0]
    xn = x / jnp.maximum(jnp.sqrt(jnp.sum(x * x, axis=1, keepdims=True)), 1e-12)
    pn = p / jnp.maximum(jnp.sqrt(jnp.sum(p * p, axis=1, keepdims=True)), 1e-12)
    logits = lax.dot_general(xn, pn, (((1,), (1,)), ((), ())),
                             preferred_element_type=jnp.float32)
    logits = logits - jnp.max(logits, axis=1, keepdims=True)
    ex = jnp.exp(logits)
    att = ex / jnp.sum(ex, axis=1, keepdims=True)
    pf = x - jnp.dot(att, p, preferred_element_type=jnp.float32)
    gr = jax.nn.sigmoid(jnp.dot(pf, gate_ref[0], preferred_element_type=jnp.float32)
                        + bias_ref[0])
    o_ref[0] = gr * x + (1.0 - gr) * pf


def _tc_tail(outs2, gate2, proxy2, bias2):
    _, N, F3 = outs2.shape
    P = proxy2.shape[1]
    BN = 1000
    nb = N // BN
    return pl.pallas_call(
        _tail_body,
        grid=(2, nb),
        in_specs=[
            pl.BlockSpec((1, BN, F3), lambda d, i: (d, i, 0)),
            pl.BlockSpec((1, F3, F3), lambda d, i: (d, 0, 0)),
            pl.BlockSpec((1, P, F3), lambda d, i: (d, 0, 0)),
            pl.BlockSpec((1, 1, F3), lambda d, i: (d, 0, 0)),
        ],
        out_specs=pl.BlockSpec((1, BN, F3), lambda d, i: (d, i, 0)),
        out_shape=jax.ShapeDtypeStruct((2, N, F3), jnp.float32),
    )(outs2, gate2, proxy2, bias2)


def _sc_body(NP, E, R4, D, C,
             src2_h, dst2_h, rel2_h, tcat_h, rhat_h, hflat_h, out_h,
             acc, stats, srcs, dsts, rels, idxb, idxa, idxc, rela, relc, hbuf,
             xb, rb, statb, statdr, zb, htabA, htabB, gsem, rsem):
    _sc_impl(NP, E, R4, D, C, lax.axis_index("c"), lax.axis_index("s"),
             plsc.subcore_barrier,
             src2_h, dst2_h, rel2_h, tcat_h, rhat_h, hflat_h, out_h,
             acc, stats, srcs, dsts, rels, idxb, idxa, idxc, rela, relc, hbuf,
             xb, rb, statb, statdr, zb, htabA, htabB, gsem, rsem)


def _sc_impl(NP, E, R4, D, C, cid, sid, barrier,
             src2_h, dst2_h, rel2_h, tcat_h, rhat_h, hflat_h, out_h,
             acc, stats, srcs, dsts, rels, idxb, idxa, idxc, rela, relc, hbuf,
             xb, rb, statb, statdr, zb, htabA, htabB, gsem, rsem):
    SUP = 5
    ept = E // NS
    nchunk = ept // C
    nsup = nchunk // SUP
    npt = NP // NS
    RC = 64
    nrc = npt // RC
    NG = C // LANES
    row_lo = sid * nchunk
    zero16 = jnp.zeros((LANES,), jnp.float32)
    iota16 = lax.iota(jnp.int32, LANES)

    _copy(hflat_h.at[pl.ds(pl.multiple_of(cid * 2 * R4, 8), R4)], htabA)
    _copy(hflat_h.at[pl.ds(pl.multiple_of(cid * 2 * R4 + R4, 8), R4)], htabB)

    def _zero_rows(ref, rows, width):
        def row(i, _):
            for j in range(width // LANES):
                ref[i, pl.ds(j * LANES, LANES)] = zero16
            return 0
        _fori(0, rows, row, 0)

    _zero_rows(zb, 32, D)
    _zero_rows(statdr, RC, LANES)
    for rc in range(nrc):
        rbase = pl.multiple_of(sid * npt + rc * RC, 8)
        for q in range(RC // 32):
            _copy(zb.at[pl.ds(0, 32)], acc.at[pl.ds(rbase + q * 32, 32)])
        _copy(statdr.at[pl.ds(0, RC)], stats.at[pl.ds(rbase, RC)])
    one_col = jnp.where(iota16 == 0, 1.0, 0.0)

    def _statb_row(i, _):
        statb[i, pl.ds(0, LANES)] = one_col
        return 0
    _fori(0, C, _statb_row, 0)
    barrier()

    def _load_sup(sc, need_src):
        rb8 = row_lo + sc * SUP
        if need_src:
            _copy(src2_h.at[pl.ds(rb8, SUP)], srcs)
        _copy(dst2_h.at[pl.ds(rb8, SUP)], dsts)
        _copy(rel2_h.at[pl.ds(rb8, SUP)], rels)

    ccast = jnp.full((LANES,), 0, jnp.int32) + cid

    def _pass1_sup(sc, _):
        _load_sup(sc, True)

        def chunk(j, _):
            for g in range(NG):
                sl = pl.ds(g * LANES, LANES)
                rel16 = rels[j, sl]
                ha = plsc.load_gather(htabA, [rel16])
                hb = plsc.load_gather(htabB, [rel16])
                rows = iota16 + (g * LANES)
                plsc.store_scatter(statb, [rows, jnp.full((LANES,), 1, jnp.int32)], ha)
                plsc.store_scatter(statb, [rows, jnp.full((LANES,), 2, jnp.int32)], hb)
                sv = srcs[j, sl]
                rv = rel16 + NP
                idxb[sl] = sv + ccast * (rv - sv)
            gx = _gather_rows(tcat_h, idxb, xb, gsem)
            _scatter_add_rows(statb, stats, dsts.at[j])
            gx.wait()
            _scatter_add_rows(xb, acc, dsts.at[j])
            return 0
        _fori(0, SUP, chunk, 0)
        return 0

    def _depth_sup(htab, tbase, sc, _):
        _load_sup(sc, True)

        HG = 3
        def chunk(j, _):
            for g in range(NG):
                sl = pl.ds(g * LANES, LANES)
                gsl = pl.ds((g * LANES) % (HG * LANES), LANES)
                ib = idxa if g < HG else idxc
                rf = rela if g < HG else relc
                ib[gsl] = srcs[j, sl] + tbase
                rf[gsl] = rels[j, sl]
                hbuf[sl] = plsc.load_gather(htab, [rels[j, sl]])
            gxA = _gather_rows(out_h, idxa, xb.at[pl.ds(0, HG * LANES)], gsem)
            grA = _gather_rows(rhat_h, rela, rb.at[pl.ds(0, HG * LANES)], rsem)
            gxB = _gather_rows(out_h, idxc,
                               xb.at[pl.ds(HG * LANES, C - HG * LANES)], gsem)
            grB = _gather_rows(rhat_h, relc,
                               rb.at[pl.ds(HG * LANES, C - HG * LANES)], rsem)
            gxA.wait()
            grA.wait()

            def group(gi, _):
                @pl.when(gi == HG)
                def _wait_second_half():
                    gxB.wait()
                    grB.wait()
                a16 = hbuf[pl.ds(gi * LANES, LANES)]
                for l in range(LANES):
                    i = gi * LANES + l
                    a = a16[l]
                    dotv = xb[i, pl.ds(0, LANES)] * rb[i, pl.ds(0, LANES)]
                    for jj in range(1, D // LANES):
                        sl = pl.ds(jj * LANES, LANES)
                        dotv = dotv + xb[i, sl] * rb[i, sl]
                    b = -2.0 * a * jnp.sum(dotv)
                    for jj in range(D // LANES):
                        sl = pl.ds(jj * LANES, LANES)
                        xb[i, sl] = a * xb[i, sl] + b * rb[i, sl]
                return 0
            _fori(0, NG, group, 0)
            _scatter_add_rows(xb, acc, dsts.at[j])
            return 0
        _fori(0, SUP, chunk, 0)
        return 0

    def _drain(scol, plane_base, zero_after):
        for rc in range(nrc):
            rbase = pl.multiple_of(sid * npt + rc * RC, 8)
            _copy(acc.at[pl.ds(rbase, RC)], xb.at[pl.ds(0, RC)])
            _copy(stats.at[pl.ds(rbase, RC)], statdr.at[pl.ds(0, RC)])
            if zero_after:
                for q in range(RC // 32):
                    _copy(zb.at[pl.ds(0, 32)],
                          acc.at[pl.ds(rbase + q * 32, 32)])

            def row(i, _):
                srow = statdr[i, pl.ds(0, LANES)]
                if scol == 0:
                    gv = 1.0 / jnp.maximum(srow, 1.0)
                else:
                    gv = jnp.where(srow > 0.0, 1.0 / srow, 0.0)
                g = gv[scol]
                for j in range(D // LANES):
                    sl = pl.ds(j * LANES, LANES)
                    z = xb[i, sl] * g
                    xb[i, sl] = 1.0 - 2.0 / (jnp.exp(2.0 * z) + 1.0)
                return 0
            _fori(0, RC, row, 0)
            _copy(xb.at[pl.ds(0, RC)],
                  out_h.at[pl.ds(pl.multiple_of(plane_base + rbase, 8), RC)])
        barrier()

    _fori(0, nsup, _pass1_sup, 0)
    barrier()
    plane0 = pl.multiple_of(cid * (3 * NP), 8)
    _drain(0, plane0, True)

    _fori(0, nsup, functools.partial(_depth_sup, htabA, plane0), 0)
    barrier()
    _drain(1, plane0 + NP, True)

    _fori(0, nsup, functools.partial(_depth_sup, htabB, plane0 + NP), 0)
    barrier()
    _drain(2, plane0 + 2 * NP, False)


def _sc_passes(src2, dst2, rel2, tcat, rhat, hflat, NP, E, R4, D):
    C = src2.shape[1]
    SUP = 5
    mesh = plsc.VectorSubcoreMesh(core_axis_name="c", subcore_axis_name="s",
                                  num_cores=NC, num_subcores=NS)
    f32 = jnp.float32
    i32 = jnp.int32
    body = functools.partial(_sc_body, NP, E, R4, D, C)
    kern = pl.kernel(
        body,
        out_type=jax.ShapeDtypeStruct((6 * NP, D), f32),
        mesh=mesh,
        compiler_params=pltpu.CompilerParams(needs_layout_passes=False,
                                             use_tc_tiling_on_sc=False),
        scratch_types=[
            pltpu.VMEM_SHARED((NP, D), f32),
            pltpu.VMEM_SHARED((NP, LANES), f32),
            pltpu.VMEM((SUP, C), i32),
            pltpu.VMEM((SUP, C), i32),
            pltpu.VMEM((SUP, C), i32),
            pltpu.VMEM((C,), i32),
            pltpu.VMEM((48,), i32),
            pltpu.VMEM((32,), i32),
            pltpu.VMEM((48,), i32),
            pltpu.VMEM((32,), i32),
            pltpu.VMEM((C + LANES,), f32),
            pltpu.VMEM((C, D), f32),
            pltpu.VMEM((C, D), f32),
            pltpu.VMEM((C, LANES), f32),
            pltpu.VMEM((64, LANES), f32),
            pltpu.VMEM((32, D), f32),
            pltpu.VMEM((R4,), f32),
            pltpu.VMEM((R4,), f32),
            pltpu.SemaphoreType.DMA,
            pltpu.SemaphoreType.DMA,
        ],
    )
    return kern(src2, dst2, rel2, tcat, rhat, hflat)


def kernel(edge_index, edge_rel, ent_emb, rel_emb,
           e_gate, e_proxy, e_bias, e_attn0, e_attn1,
           r_gate, r_proxy, r_bias, r_attn0, r_attn1):
    N, D = ent_emb.shape
    R = rel_emb.shape[0]
    E = edge_rel.shape[0]
    NP = ((N + NS * 128 - 1) // (NS * 128)) * (NS * 128)
    R4 = ((R + 1023) // 1024) * 1024

    C = 80
    src2 = edge_index[0].astype(jnp.int32).reshape(E // C, C)
    dst2 = edge_index[1].astype(jnp.int32).reshape(E // C, C)
    rel2 = edge_rel.astype(jnp.int32).reshape(E // C, C)

    kmat = jnp.zeros((D, 128), jnp.float32)
    kmat = kmat.at[:, 0:1].set(e_attn0).at[:, 1:2].set(e_attn1)
    kmat = kmat.at[:, 2:3].set(r_attn0).at[:, 3:4].set(r_attn1)

    rhat, hfull = _tc_prep(rel_emb, kmat)
    hflat = jnp.zeros((4, R4), jnp.float32).at[:, :R].set(
        jnp.transpose(hfull[:, :4])).reshape(-1)

    tcat = jnp.zeros((NP + R, D), jnp.float32)
    tcat = tcat.at[:N].set(ent_emb).at[NP:].set(rel_emb)

    planes = _sc_passes(src2, dst2, rel2, tcat, rhat, hflat, NP, E, R4, D)
    planes = planes.reshape(6, NP, D)[:, :N, :]

    outs2 = jnp.stack([
        jnp.concatenate([planes[0], planes[1], planes[2]], axis=1),
        jnp.concatenate([planes[3], planes[4], planes[5]], axis=1),
    ])
    gate2 = jnp.stack([e_gate, r_gate])
    proxy2 = jnp.stack([e_proxy, r_proxy])
    bias2 = jnp.stack([e_bias, r_bias])

    res = _tc_tail(outs2, gate2, proxy2, bias2)
    return jnp.concatenate([res[0], res[1]], axis=-1)
